# Initial kernel scaffold; baseline (speedup 1.0000x reference)
import jax
import jax.numpy as jnp
from jax.experimental import pallas as pl


def kernel(nodes, edges, senders, receivers, params):
    raise NotImplementedError("write your pallas kernel here")


# speedup vs baseline: 3.7342x; 3.7342x over previous
import functools

import jax
import jax.numpy as jnp
from jax import lax
from jax.experimental import pallas as pl
from jax.experimental.pallas import tpu as pltpu
from jax.experimental.pallas import tpu_sc as plsc

_N = 10000
_E = 320000
_D = 128

_CW = 80
_CPW = 125
_K = 5
_KS = 3
_NPAD = 10112
_TILE_ROWS = _NPAD // 16

_BN = 2000
_BE = 3200

_f32 = jnp.float32


def _dot(a, b):
    return jnp.dot(a, b, preferred_element_type=_f32)


def _ln(y, scale, offset):
    m = jnp.mean(y, axis=-1, keepdims=True)
    v = jnp.mean((y - m) ** 2, axis=-1, keepdims=True)
    return (y - m) * lax.rsqrt(v + 1e-5) * scale + offset


def _rep(shape):
    return pl.BlockSpec(shape, lambda i: (0,) * len(shape))


def _blk_weights(blk):
    (l1, l2, l3) = blk["mlp"]
    return (l1["W"], l1["b"].reshape(1, _D), l2["W"], l2["b"].reshape(1, _D),
            l3["W"], l3["b"].reshape(1, -1))


def _encode_node(nodes, blk, ws, wr):
    w1, b1, w2, b2, w3, b3 = _blk_weights(blk)
    sc = blk["ln_scale"].reshape(1, _D)
    of = blk["ln_offset"].reshape(1, _D)

    def body(x_r, w1_r, b1_r, w2_r, b2_r, w3_r, b3_r, s_r, o_r, ws_r, wr_r,
             lat_r, ps_r, pr_r):
        x = x_r[...]
        h = jnp.maximum(_dot(x, w1_r[...]) + b1_r[...], 0.0)
        h = jnp.maximum(_dot(h, w2_r[...]) + b2_r[...], 0.0)
        lat = _ln(_dot(h, w3_r[...]) + b3_r[...], s_r[...], o_r[...])
        lat_r[...] = lat
        ps_r[...] = _dot(lat, ws_r[...])
        pr_r[...] = _dot(lat, wr_r[...])

    args = (nodes, w1, b1, w2, b2, w3, b3, sc, of, ws, wr)
    return pl.pallas_call(
        body,
        grid=(_N // _BN,),
        in_specs=[pl.BlockSpec((_BN, _D), lambda i: (i, 0))]
        + [_rep(a.shape) for a in args[1:]],
        out_specs=[pl.BlockSpec((_BN, _D), lambda i: (i, 0))] * 3,
        out_shape=[jax.ShapeDtypeStruct((_N, _D), _f32)] * 3,
    )(*args)


def _encode_edge(edges, blk):
    w1, b1, w2, b2, w3, b3 = _blk_weights(blk)
    sc = blk["ln_scale"].reshape(1, _D)
    of = blk["ln_offset"].reshape(1, _D)
    d_in = edges.shape[1]

    def body(x_r, w1_r, b1_r, w2_r, b2_r, w3_r, b3_r, s_r, o_r, lat_r):
        x = x_r[...]
        h = jnp.maximum(_dot(x, w1_r[...]) + b1_r[...], 0.0)
        h = jnp.maximum(_dot(h, w2_r[...]) + b2_r[...], 0.0)
        lat_r[...] = _ln(_dot(h, w3_r[...]) + b3_r[...], s_r[...], o_r[...])

    args = (edges, w1, b1, w2, b2, w3, b3, sc, of)
    return pl.pallas_call(
        body,
        grid=(_E // _BE,),
        in_specs=[pl.BlockSpec((_BE, d_in), lambda i: (i, 0))]
        + [_rep(a.shape) for a in args[1:]],
        out_specs=pl.BlockSpec((_BE, _D), lambda i: (i, 0)),
        out_shape=jax.ShapeDtypeStruct((_E, _D), _f32),
    )(*args)


def _edge_step(edge_lat, gs, gr, blk, we):
    _, b1, w2, b2, w3, b3 = _blk_weights(blk)
    sc = blk["ln_scale"].reshape(1, _D)
    of = blk["ln_offset"].reshape(1, _D)

    def body(x_r, gs_r, gr_r, we_r, b1_r, w2_r, b2_r, w3_r, b3_r, s_r, o_r,
             upd_r, new_r):
        x = x_r[...]
        h = jnp.maximum(_dot(x, we_r[...]) + gs_r[...] + gr_r[...] + b1_r[...], 0.0)
        h = jnp.maximum(_dot(h, w2_r[...]) + b2_r[...], 0.0)
        upd = _ln(_dot(h, w3_r[...]) + b3_r[...], s_r[...], o_r[...])
        upd_r[...] = upd
        new_r[...] = x + upd

    args = (edge_lat, gs, gr, we, b1, w2, b2, w3, b3, sc, of)
    return pl.pallas_call(
        body,
        grid=(_E // _BE,),
        in_specs=[pl.BlockSpec((_BE, _D), lambda i: (i, 0))] * 3
        + [_rep(a.shape) for a in args[3:]],
        out_specs=[pl.BlockSpec((_BE, _D), lambda i: (i, 0))] * 2,
        out_shape=[jax.ShapeDtypeStruct((_E, _D), _f32)] * 2,
    )(*args)


def _node_step(node_lat, r0, r1, blk, wa, wb, ws_next, wr_next):
    _, b1, w2, b2, w3, b3 = _blk_weights(blk)
    sc = blk["ln_scale"].reshape(1, _D)
    of = blk["ln_offset"].reshape(1, _D)

    def body(x_r, r0_r, r1_r, wa_r, wb_r, b1_r, w2_r, b2_r, w3_r, b3_r, s_r,
             o_r, wsn_r, wrn_r, new_r, ps_r, pr_r):
        x = x_r[...]
        r = r0_r[...] + r1_r[...]
        h = jnp.maximum(_dot(x, wa_r[...]) + _dot(r, wb_r[...]) + b1_r[...], 0.0)
        h = jnp.maximum(_dot(h, w2_r[...]) + b2_r[...], 0.0)
        new = x + _ln(_dot(h, w3_r[...]) + b3_r[...], s_r[...], o_r[...])
        new_r[...] = new
        ps_r[...] = _dot(new, wsn_r[...])
        pr_r[...] = _dot(new, wrn_r[...])

    args = (node_lat, r0, r1, wa, wb, b1, w2, b2, w3, b3, sc, of, ws_next,
            wr_next)
    return pl.pallas_call(
        body,
        grid=(_N // _BN,),
        in_specs=[pl.BlockSpec((_BN, _D), lambda i: (i, 0))] * 3
        + [_rep(a.shape) for a in args[3:]],
        out_specs=[pl.BlockSpec((_BN, _D), lambda i: (i, 0))] * 3,
        out_shape=[jax.ShapeDtypeStruct((_N, _D), _f32)] * 3,
    )(*args)


def _node_last_step(node_lat, r0, r1, blk, wa, wb, dec):
    _, b1, w2, b2, w3, b3 = _blk_weights(blk)
    sc = blk["ln_scale"].reshape(1, _D)
    of = blk["ln_offset"].reshape(1, _D)
    d1, d2, d3 = dec
    dw1, db1 = d1["W"], d1["b"].reshape(1, _D)
    dw2, db2 = d2["W"], d2["b"].reshape(1, _D)
    dw3 = jnp.pad(d3["W"], ((0, 0), (0, _D - d3["W"].shape[1])))
    db3 = jnp.pad(d3["b"], (0, _D - d3["b"].shape[0])).reshape(1, _D)

    def body(x_r, r0_r, r1_r, wa_r, wb_r, b1_r, w2_r, b2_r, w3_r, b3_r, s_r,
             o_r, dw1_r, db1_r, dw2_r, db2_r, dw3_r, db3_r, out_r):
        x = x_r[...]
        r = r0_r[...] + r1_r[...]
        h = jnp.maximum(_dot(x, wa_r[...]) + _dot(r, wb_r[...]) + b1_r[...], 0.0)
        h = jnp.maximum(_dot(h, w2_r[...]) + b2_r[...], 0.0)
        new = x + _ln(_dot(h, w3_r[...]) + b3_r[...], s_r[...], o_r[...])
        g = jnp.maximum(_dot(new, dw1_r[...]) + db1_r[...], 0.0)
        g = jnp.maximum(_dot(g, dw2_r[...]) + db2_r[...], 0.0)
        out_r[...] = _dot(g, dw3_r[...]) + db3_r[...]

    args = (node_lat, r0, r1, wa, wb, b1, w2, b2, w3, b3, sc, of, dw1, db1,
            dw2, db2, dw3, db3)
    return pl.pallas_call(
        body,
        grid=(_N // _BN,),
        in_specs=[pl.BlockSpec((_BN, _D), lambda i: (i, 0))] * 3
        + [_rep(a.shape) for a in args[3:]],
        out_specs=pl.BlockSpec((_BN, _D), lambda i: (i, 0)),
        out_shape=jax.ShapeDtypeStruct((_N, _D), _f32),
    )(*args)


@functools.lru_cache(maxsize=None)
def _gather_fn():
    mesh = plsc.VectorSubcoreMesh(core_axis_name="c", subcore_axis_name="s")

    @functools.partial(
        pl.kernel,
        out_type=(jax.ShapeDtypeStruct((_E, _D), _f32),
                  jax.ShapeDtypeStruct((_E, _D), _f32)),
        mesh=mesh,
        scratch_types=[
            pltpu.VMEM((_CPW, _CW), jnp.int32),
            pltpu.VMEM((_K * _CW, _D), _f32),
            pltpu.SemaphoreType.DMA,
            pltpu.SemaphoreType.DMA,
        ],
    )
    def gather(ps_h, pr_h, send_h, recv_h, gs_h, gr_h, idx_v, buf_v, gsem,
               ssem):
        c = lax.axis_index("c")
        s = lax.axis_index("s")
        w = s * 2 + c
        base = w * _CPW
        for tab_h, ind_h, out_h in ((ps_h, send_h, gs_h), (pr_h, recv_h, gr_h)):
            pltpu.sync_copy(ind_h.at[w], idx_v)

            def batch(g, carry):
                j0 = g * _K
                gds = [
                    pltpu.async_copy(tab_h.at[idx_v.at[j0 + b]],
                                     buf_v.at[pl.ds(b * _CW, _CW)], gsem)
                    for b in range(_K)
                ]
                for d_ in gds:
                    d_.wait()
                sds = [
                    pltpu.async_copy(buf_v.at[pl.ds(b * _CW, _CW)],
                                     out_h.at[pl.ds((base + j0 + b) * _CW, _CW)],
                                     ssem)
                    for b in range(_K)
                ]
                for d_ in sds:
                    d_.wait()
                return carry

            lax.fori_loop(0, _CPW // _K, batch, 0)

    return gather


@functools.lru_cache(maxsize=None)
def _scatter_fn():
    mesh = plsc.VectorSubcoreMesh(core_axis_name="c", subcore_axis_name="s")

    @functools.partial(
        pl.kernel,
        out_type=jax.ShapeDtypeStruct((2, _NPAD, _D), _f32),
        mesh=mesh,
        scratch_types=[
            pltpu.VMEM((_CPW, _CW), jnp.int32),
            pltpu.VMEM((_KS * _CW, _D), _f32),
            pltpu.VMEM_SHARED((_NPAD, _D), _f32),
            pltpu.SemaphoreType.DMA,
        ],
    )
    def scatter(eupd_h, recv_h, zer_h, parts_h, idx_v, buf_v, acc_sh, lsem):
        c = lax.axis_index("c")
        s = lax.axis_index("s")
        w = s * 2 + c
        base = w * _CPW
        pltpu.sync_copy(zer_h, acc_sh.at[pl.ds(s * _TILE_ROWS, _TILE_ROWS)])
        pltpu.sync_copy(recv_h.at[w], idx_v)
        plsc.subcore_barrier()

        def batch(g, carry):
            j0 = g * _KS
            pltpu.async_copy(
                eupd_h.at[pl.ds((base + j0) * _CW, _KS * _CW)], buf_v,
                lsem).wait()
            for b in range(_KS):
                pltpu.sync_copy(buf_v.at[pl.ds(b * _CW, _CW)],
                                acc_sh.at[idx_v.at[j0 + b]], add=True)
            return carry

        lax.fori_loop(0, _CPW // _KS, batch, 0)
        jt = (_CPW // _KS) * _KS
        nt = _CPW - jt
        pltpu.async_copy(eupd_h.at[pl.ds((base + jt) * _CW, nt * _CW)],
                         buf_v.at[pl.ds(0, nt * _CW)], lsem).wait()
        for b in range(nt):
            pltpu.sync_copy(buf_v.at[pl.ds(b * _CW, _CW)],
                            acc_sh.at[idx_v.at[jt + b]], add=True)
        plsc.subcore_barrier()
        pltpu.sync_copy(acc_sh.at[pl.ds(s * _TILE_ROWS, _TILE_ROWS)],
                        parts_h.at[c, pl.ds(s * _TILE_ROWS, _TILE_ROWS)])

    return scatter


def kernel(nodes, edges, senders, receivers, params):
    send2 = senders.reshape(32, _CPW, _CW)
    recv2 = receivers.reshape(32, _CPW, _CW)
    zer = jnp.zeros((_TILE_ROWS, _D), _f32)

    proc = params["proc"]
    edge_w1 = [st["edge"]["mlp"][0]["W"] for st in proc]
    node_w1 = [st["node"]["mlp"][0]["W"] for st in proc]
    we = [w[:_D] for w in edge_w1]
    ws = [w[_D:2 * _D] for w in edge_w1]
    wr = [w[2 * _D:] for w in edge_w1]
    wa = [w[:_D] for w in node_w1]
    wb = [w[_D:] for w in node_w1]

    node_lat, ps, pr = _encode_node(nodes, params["enc_node"], ws[0], wr[0])
    edge_lat = _encode_edge(edges, params["enc_edge"])

    gather = _gather_fn()
    scatter = _scatter_fn()

    for i in range(len(proc)):
        gs, gr = gather(ps, pr, send2, recv2)
        e_upd, edge_lat = _edge_step(edge_lat, gs, gr, proc[i]["edge"], we[i])
        parts = scatter(e_upd, recv2, zer)
        parts = parts[:, :_N]
        if i + 1 < len(proc):
            node_lat, ps, pr = _node_step(node_lat, parts[0], parts[1],
                                          proc[i]["node"], wa[i], wb[i],
                                          ws[i + 1], wr[i + 1])
        else:
            out = _node_last_step(node_lat, parts[0], parts[1],
                                  proc[i]["node"], wa[i], wb[i],
                                  params["dec_node"]["mlp"])
    return out[:, :3]
